# Initial kernel scaffold; baseline (speedup 1.0000x reference)
import jax
import jax.numpy as jnp
from jax.experimental import pallas as pl


def kernel(x, tables, proj_w, proj_b):
    raise NotImplementedError("write your pallas kernel here")


# speedup vs baseline: 4.1480x; 4.1480x over previous
import functools

import jax
import jax.numpy as jnp
from jax import lax
from jax.experimental import pallas as pl
from jax.experimental.pallas import tpu as pltpu
from jax.experimental.pallas import tpu_sc as plsc

NUM_FIELDS = 8
VOCAB = 100000
EMB = 64
PROJ = 128

NC, NS = 2, 16
NW = NC * NS
CH = 128
GRP = 4
HALF = CH * GRP


def _sc_gather(flat_table, idx2d, total_rows):
    rpw = total_rows // NW
    nhalf = rpw // HALF

    mesh = plsc.VectorSubcoreMesh(core_axis_name="c", subcore_axis_name="s")

    @functools.partial(
        pl.kernel,
        out_type=jax.ShapeDtypeStruct((total_rows, EMB), jnp.float32),
        mesh=mesh,
        scratch_types=[
            pltpu.VMEM((GRP, CH), jnp.int32),
            pltpu.VMEM((HALF, EMB), jnp.float32),
            pltpu.SemaphoreType.DMA,
        ],
        compiler_params=pltpu.CompilerParams(use_tc_tiling_on_sc=False),
    )
    def k(tab_hbm, idx_hbm, out_hbm, idx_v, rows_v, gsem):
        wid = lax.axis_index("s") * NC + lax.axis_index("c")
        row_base = wid * rpw
        idx_base = wid * (rpw // CH)

        def half_body(hi, carry):
            off = row_base + hi * HALF
            pltpu.sync_copy(idx_hbm.at[pl.ds(idx_base + hi * GRP, GRP)], idx_v)
            for j in range(GRP):
                pltpu.async_copy(
                    tab_hbm.at[idx_v.at[j]],
                    rows_v.at[pl.ds(j * CH, CH)],
                    gsem,
                )
            for j in range(GRP):
                pltpu.make_async_copy(
                    tab_hbm.at[idx_v.at[j]],
                    rows_v.at[pl.ds(j * CH, CH)],
                    gsem,
                ).wait()
            pltpu.sync_copy(rows_v, out_hbm.at[pl.ds(off, HALF)])
            return carry

        lax.fori_loop(0, nhalf, half_body, 0)

    return k(flat_table, idx2d)


def _mm_body(x_ref, w_ref, b_ref, o_ref):
    o_ref[...] = (
        jnp.dot(x_ref[...], w_ref[...], preferred_element_type=jnp.float32)
        + b_ref[...]
    )


def _tc_project(cat, proj_w, proj_b2d, bm):
    n = cat.shape[0]
    return pl.pallas_call(
        _mm_body,
        grid=(n // bm,),
        in_specs=[
            pl.BlockSpec((bm, NUM_FIELDS * EMB), lambda i: (i, 0)),
            pl.BlockSpec((NUM_FIELDS * EMB, PROJ), lambda i: (0, 0)),
            pl.BlockSpec((1, PROJ), lambda i: (0, 0)),
        ],
        out_specs=pl.BlockSpec((bm, PROJ), lambda i: (i, 0)),
        out_shape=jax.ShapeDtypeStruct((n, PROJ), jnp.float32),
    )(cat, proj_w, proj_b2d)


def kernel(x, tables, proj_w, proj_b):
    b, l, nf = x.shape
    n = b * l
    total_rows = n * nf
    offsets = (jnp.arange(nf, dtype=jnp.int32) * VOCAB).reshape(1, 1, nf)
    idx = (x.astype(jnp.int32) + offsets).reshape(total_rows // CH, CH)
    flat_table = tables.reshape(nf * VOCAB, EMB)
    cat = _sc_gather(flat_table, idx, total_rows)
    cat2 = cat.reshape(n, nf * EMB)
    out = _tc_project(cat2, proj_w, proj_b.reshape(1, PROJ), 1024)
    return out.reshape(b, l, PROJ)
